# baseline (speedup 1.0000x reference)
import functools

import jax
import jax.numpy as jnp
from jax import lax
from jax.experimental import pallas as pl
from jax.experimental.pallas import tpu as pltpu
from jax.experimental.pallas import tpu_sc as plsc

_VOCAB = 1000000
_EMSIZE = 64
_PADE = 128
_BATCH = 4096
_SEQLEN = 200

_N = _BATCH * _SEQLEN

_info = plsc.get_sparse_core_info()
_NC, _NS = _info.num_cores, _info.num_subcores
_NW = _NC * _NS
_RPW = _N // _NW

_K = 128
_NB = 4
_CPW = _RPW // _K


def _make_gather():
    mesh = plsc.VectorSubcoreMesh(core_axis_name="c", subcore_axis_name="s")

    @functools.partial(
        pl.kernel,
        mesh=mesh,
        out_type=jax.ShapeDtypeStruct((_N, _PADE), jnp.float32),
        scratch_types=[
            pltpu.VMEM((_RPW,), jnp.int32),
            [pltpu.VMEM((_K, _PADE), jnp.float32) for _ in range(_NB)],
            pltpu.SemaphoreType.DMA((_NB,)),
            pltpu.SemaphoreType.DMA((_NB,)),
        ],
        compiler_params=pltpu.CompilerParams(use_tc_tiling_on_sc=True),
    )
    def gather_kernel(table_hbm, idx_hbm, out_hbm, idx_v, bufs, gsem, wsem):
        wid = lax.axis_index("s") * _NC + lax.axis_index("c")
        row0 = wid * _RPW
        pltpu.sync_copy(idx_hbm.at[pl.ds(row0, _RPW)], idx_v)

        for b in range(_NB):
            pltpu.async_copy(
                table_hbm.at[idx_v.at[pl.ds(b * _K, _K)]],
                bufs[b],
                gsem.at[b],
            )

        def body(i, carry):
            for b in range(_NB):
                g = i * _NB + b
                pltpu.make_async_copy(
                    table_hbm.at[idx_v.at[pl.ds(g * _K, _K)]],
                    bufs[b],
                    gsem.at[b],
                ).wait()
                pltpu.async_copy(
                    bufs[b],
                    out_hbm.at[pl.ds(row0 + g * _K, _K)],
                    wsem.at[b],
                )
            for b in range(_NB):
                gnext = jnp.minimum((i + 1) * _NB + b, _CPW - 1)
                pltpu.make_async_copy(
                    bufs[b], out_hbm.at[pl.ds(0, _K)], wsem.at[b]
                ).wait()
                pltpu.async_copy(
                    table_hbm.at[idx_v.at[pl.ds(gnext * _K, _K)]],
                    bufs[b],
                    gsem.at[b],
                )
            return carry

        lax.fori_loop(0, _CPW // _NB, body, 0)

        for b in range(_NB):
            pltpu.make_async_copy(
                table_hbm.at[idx_v.at[pl.ds(0, _K)]],
                bufs[b],
                gsem.at[b],
            ).wait()

    return gather_kernel


_gather = _make_gather()


_TBC = 1920


def _make_padder():
    def body(in_ref, out_ref):
        r = jax.lax.broadcasted_iota(jnp.int32, (_EMSIZE, _EMSIZE), 0)
        c = jax.lax.broadcasted_iota(jnp.int32, (_EMSIZE, _EMSIZE), 1)
        eye = (r == c).astype(jnp.float32)
        x = in_ref[...]
        y = jax.lax.dot_general(
            x,
            eye,
            (((0,), (0,)), ((), ())),
            precision=jax.lax.Precision.HIGHEST,
            preferred_element_type=jnp.float32,
        )
        out_ref[:, :_EMSIZE] = y

    return pl.pallas_call(
        body,
        grid=((_VOCAB + _TBC - 1) // _TBC,),
        in_specs=[pl.BlockSpec((_EMSIZE, _TBC), lambda i: (0, i))],
        out_specs=pl.BlockSpec((_TBC, _PADE), lambda i: (i, 0)),
        out_shape=jax.ShapeDtypeStruct((_VOCAB, _PADE), jnp.float32),
    )


_padder = _make_padder()


def kernel(sequence, table):
    idx = sequence.astype(jnp.int32).reshape(_N)
    table_pad = _padder(table.T)
    out = _gather(table_pad, idx)
    return out[:, :_EMSIZE].reshape(_BATCH, _SEQLEN, _EMSIZE)


# speedup vs baseline: 1.3326x; 1.3326x over previous
import functools

import jax
import jax.numpy as jnp
from jax import lax
from jax.experimental import pallas as pl
from jax.experimental.pallas import tpu as pltpu
from jax.experimental.pallas import tpu_sc as plsc

_VOCAB = 1000000
_EMSIZE = 64
_PADE = 128
_BATCH = 4096
_SEQLEN = 200

_N = _BATCH * _SEQLEN

_info = plsc.get_sparse_core_info()
_NC, _NS = _info.num_cores, _info.num_subcores
_NW = _NC * _NS
_RPW = _N // _NW

_K = 128
_NB = 4
_CPW = _RPW // _K


def _make_gather():
    mesh = plsc.VectorSubcoreMesh(core_axis_name="c", subcore_axis_name="s")

    @functools.partial(
        pl.kernel,
        mesh=mesh,
        out_type=jax.ShapeDtypeStruct((_N, _PADE), jnp.float32),
        scratch_types=[
            pltpu.VMEM((_RPW,), jnp.int32),
            [pltpu.VMEM((_K, _PADE), jnp.float32) for _ in range(_NB)],
            pltpu.SemaphoreType.DMA((_NB,)),
            pltpu.SemaphoreType.DMA((_NB,)),
        ],
        compiler_params=pltpu.CompilerParams(use_tc_tiling_on_sc=True),
    )
    def gather_kernel(table_hbm, idx_hbm, out_hbm, idx_v, bufs, gsem, wsem):
        wid = lax.axis_index("s") * _NC + lax.axis_index("c")
        row0 = wid * _RPW
        pltpu.sync_copy(idx_hbm.at[pl.ds(row0, _RPW)], idx_v)

        for b in range(_NB):
            pltpu.async_copy(
                table_hbm.at[idx_v.at[pl.ds(b * _K, _K)]],
                bufs[b],
                gsem.at[b],
            )

        def body(i, carry):
            for b in range(_NB):
                g = i * _NB + b
                pltpu.make_async_copy(
                    table_hbm.at[idx_v.at[pl.ds(g * _K, _K)]],
                    bufs[b],
                    gsem.at[b],
                ).wait()
                pltpu.async_copy(
                    bufs[b],
                    out_hbm.at[pl.ds(row0 + g * _K, _K)],
                    wsem.at[b],
                )
            for b in range(_NB):
                gnext = jnp.minimum((i + 1) * _NB + b, _CPW - 1)
                pltpu.make_async_copy(
                    bufs[b], out_hbm.at[pl.ds(0, _K)], wsem.at[b]
                ).wait()
                pltpu.async_copy(
                    table_hbm.at[idx_v.at[pl.ds(gnext * _K, _K)]],
                    bufs[b],
                    gsem.at[b],
                )
            return carry

        lax.fori_loop(0, _CPW // _NB, body, 0)

        for b in range(_NB):
            pltpu.make_async_copy(
                table_hbm.at[idx_v.at[pl.ds(0, _K)]],
                bufs[b],
                gsem.at[b],
            ).wait()

    return gather_kernel


_gather = _make_gather()


_TBC = 3840


def _make_padder():
    def body(in_ref, out_ref):
        out_ref[:, :_EMSIZE] = jnp.transpose(in_ref[...])

    return pl.pallas_call(
        body,
        grid=((_VOCAB + _TBC - 1) // _TBC,),
        in_specs=[pl.BlockSpec((_EMSIZE, _TBC), lambda i: (0, i))],
        out_specs=pl.BlockSpec((_TBC, _PADE), lambda i: (i, 0)),
        out_shape=jax.ShapeDtypeStruct((_VOCAB, _PADE), jnp.float32),
    )


_padder = _make_padder()


def kernel(sequence, table):
    idx = sequence.astype(jnp.int32).reshape(_N)
    table_pad = _padder(table.T)
    out = _gather(table_pad, idx)
    return out[:, :_EMSIZE].reshape(_BATCH, _SEQLEN, _EMSIZE)


# speedup vs baseline: 1.4685x; 1.1020x over previous
import functools

import jax
import jax.numpy as jnp
from jax import lax
from jax.experimental import pallas as pl
from jax.experimental.pallas import tpu as pltpu
from jax.experimental.pallas import tpu_sc as plsc

_VOCAB = 1000000
_EMSIZE = 64
_PADE = 128
_BATCH = 4096
_SEQLEN = 200

_N = _BATCH * _SEQLEN

_info = plsc.get_sparse_core_info()
_NC, _NS = _info.num_cores, _info.num_subcores
_NW = _NC * _NS
_RPW = _N // _NW

_K = 128
_NB = 4
_CPW = _RPW // _K


def _make_gather():
    mesh = plsc.VectorSubcoreMesh(core_axis_name="c", subcore_axis_name="s")

    @functools.partial(
        pl.kernel,
        mesh=mesh,
        out_type=jax.ShapeDtypeStruct((_N, _PADE), jnp.float32),
        scratch_types=[
            pltpu.VMEM((_RPW,), jnp.int32),
            [pltpu.VMEM((_K, _PADE), jnp.float32) for _ in range(_NB)],
            pltpu.SemaphoreType.DMA((_NB,)),
            pltpu.SemaphoreType.DMA((_NB,)),
        ],
        compiler_params=pltpu.CompilerParams(use_tc_tiling_on_sc=True),
    )
    def gather_kernel(table_hbm, idx_hbm, out_hbm, idx_v, bufs, gsem, wsem):
        wid = lax.axis_index("s") * _NC + lax.axis_index("c")
        row0 = wid * _RPW
        pltpu.sync_copy(idx_hbm.at[pl.ds(row0, _RPW)], idx_v)

        for b in range(_NB):
            pltpu.async_copy(
                table_hbm.at[idx_v.at[pl.ds(b * _K, _K)]],
                bufs[b],
                gsem.at[b],
            )

        def body(i, carry):
            for b in range(_NB):
                g = i * _NB + b
                pltpu.make_async_copy(
                    table_hbm.at[idx_v.at[pl.ds(g * _K, _K)]],
                    bufs[b],
                    gsem.at[b],
                ).wait()
                pltpu.async_copy(
                    bufs[b],
                    out_hbm.at[pl.ds(row0 + g * _K, _K)],
                    wsem.at[b],
                )
            for b in range(_NB):
                gnext = jnp.minimum((i + 1) * _NB + b, _CPW - 1)
                pltpu.make_async_copy(
                    bufs[b], out_hbm.at[pl.ds(0, _K)], wsem.at[b]
                ).wait()
                pltpu.async_copy(
                    table_hbm.at[idx_v.at[pl.ds(gnext * _K, _K)]],
                    bufs[b],
                    gsem.at[b],
                )
            return carry

        lax.fori_loop(0, _CPW // _NB, body, 0)

        for b in range(_NB):
            pltpu.make_async_copy(
                table_hbm.at[idx_v.at[pl.ds(0, _K)]],
                bufs[b],
                gsem.at[b],
            ).wait()

    return gather_kernel


_gather = _make_gather()


_TBC = 7680


def _make_padder():
    def body(in_ref, out_ref):
        out_ref[:, :_EMSIZE] = jnp.transpose(in_ref[...])

    return pl.pallas_call(
        body,
        grid=((_VOCAB + _TBC - 1) // _TBC,),
        in_specs=[pl.BlockSpec((_EMSIZE, _TBC), lambda i: (0, i))],
        out_specs=pl.BlockSpec((_TBC, _PADE), lambda i: (i, 0)),
        out_shape=jax.ShapeDtypeStruct((_VOCAB, _PADE), jnp.float32),
    )


_padder = _make_padder()


def kernel(sequence, table):
    idx = sequence.astype(jnp.int32).reshape(_N)
    table_pad = _padder(table.T)
    out = _gather(table_pad, idx)
    return out[:, :_EMSIZE].reshape(_BATCH, _SEQLEN, _EMSIZE)


# speedup vs baseline: 1.5158x; 1.0322x over previous
import functools

import jax
import jax.numpy as jnp
from jax import lax
from jax.experimental import pallas as pl
from jax.experimental.pallas import tpu as pltpu
from jax.experimental.pallas import tpu_sc as plsc

_VOCAB = 1000000
_EMSIZE = 64
_PADE = 128
_BATCH = 4096
_SEQLEN = 200

_N = _BATCH * _SEQLEN

_info = plsc.get_sparse_core_info()
_NC, _NS = _info.num_cores, _info.num_subcores
_NW = _NC * _NS
_RPW = _N // _NW

_K = 128
_NB = 4
_CPW = _RPW // _K


def _make_gather():
    mesh = plsc.VectorSubcoreMesh(core_axis_name="c", subcore_axis_name="s")

    @functools.partial(
        pl.kernel,
        mesh=mesh,
        out_type=jax.ShapeDtypeStruct((_N, _PADE), jnp.float32),
        scratch_types=[
            pltpu.VMEM((_RPW,), jnp.int32),
            [pltpu.VMEM((_K, _PADE), jnp.float32) for _ in range(_NB)],
            pltpu.SemaphoreType.DMA((_NB,)),
            pltpu.SemaphoreType.DMA((_NB,)),
        ],
        compiler_params=pltpu.CompilerParams(use_tc_tiling_on_sc=True),
    )
    def gather_kernel(table_hbm, idx_hbm, out_hbm, idx_v, bufs, gsem, wsem):
        wid = lax.axis_index("s") * _NC + lax.axis_index("c")
        row0 = wid * _RPW
        pltpu.sync_copy(idx_hbm.at[pl.ds(row0, _RPW)], idx_v)

        for b in range(_NB):
            pltpu.async_copy(
                table_hbm.at[idx_v.at[pl.ds(b * _K, _K)]],
                bufs[b],
                gsem.at[b],
            )

        def body(i, carry):
            for b in range(_NB):
                g = i * _NB + b
                pltpu.make_async_copy(
                    table_hbm.at[idx_v.at[pl.ds(g * _K, _K)]],
                    bufs[b],
                    gsem.at[b],
                ).wait()
                pltpu.async_copy(
                    bufs[b],
                    out_hbm.at[pl.ds(row0 + g * _K, _K)],
                    wsem.at[b],
                )
            for b in range(_NB):
                gnext = jnp.minimum((i + 1) * _NB + b, _CPW - 1)
                pltpu.make_async_copy(
                    bufs[b], out_hbm.at[pl.ds(0, _K)], wsem.at[b]
                ).wait()
                pltpu.async_copy(
                    table_hbm.at[idx_v.at[pl.ds(gnext * _K, _K)]],
                    bufs[b],
                    gsem.at[b],
                )
            return carry

        lax.fori_loop(0, _CPW // _NB, body, 0)

        for b in range(_NB):
            pltpu.make_async_copy(
                table_hbm.at[idx_v.at[pl.ds(0, _K)]],
                bufs[b],
                gsem.at[b],
            ).wait()

    return gather_kernel


_gather = _make_gather()


_TBC = 15360


def _make_padder():
    def body(in_ref, out_ref):
        out_ref[:, :_EMSIZE] = jnp.transpose(in_ref[...])

    return pl.pallas_call(
        body,
        grid=((_VOCAB + _TBC - 1) // _TBC,),
        in_specs=[pl.BlockSpec((_EMSIZE, _TBC), lambda i: (0, i))],
        out_specs=pl.BlockSpec((_TBC, _PADE), lambda i: (i, 0)),
        out_shape=jax.ShapeDtypeStruct((_VOCAB, _PADE), jnp.float32),
    )


_padder = _make_padder()


def kernel(sequence, table):
    idx = sequence.astype(jnp.int32).reshape(_N)
    table_pad = _padder(table.T)
    out = _gather(table_pad, idx)
    return out[:, :_EMSIZE].reshape(_BATCH, _SEQLEN, _EMSIZE)


# speedup vs baseline: 1.5286x; 1.0085x over previous
import functools

import jax
import jax.numpy as jnp
from jax import lax
from jax.experimental import pallas as pl
from jax.experimental.pallas import tpu as pltpu
from jax.experimental.pallas import tpu_sc as plsc

_VOCAB = 1000000
_EMSIZE = 64
_PADE = 128
_BATCH = 4096
_SEQLEN = 200

_N = _BATCH * _SEQLEN

_info = plsc.get_sparse_core_info()
_NC, _NS = _info.num_cores, _info.num_subcores
_NW = _NC * _NS
_RPW = _N // _NW

_K = 128
_NB = 4
_CPW = _RPW // _K


def _make_gather():
    mesh = plsc.VectorSubcoreMesh(core_axis_name="c", subcore_axis_name="s")

    @functools.partial(
        pl.kernel,
        mesh=mesh,
        out_type=jax.ShapeDtypeStruct((_N, _PADE), jnp.float32),
        scratch_types=[
            pltpu.VMEM((_RPW,), jnp.int32),
            [pltpu.VMEM((_K, _PADE), jnp.float32) for _ in range(_NB)],
            pltpu.SemaphoreType.DMA((_NB,)),
            pltpu.SemaphoreType.DMA((_NB,)),
        ],
        compiler_params=pltpu.CompilerParams(use_tc_tiling_on_sc=True),
    )
    def gather_kernel(table_hbm, idx_hbm, out_hbm, idx_v, bufs, gsem, wsem):
        wid = lax.axis_index("s") * _NC + lax.axis_index("c")
        row0 = wid * _RPW
        pltpu.sync_copy(idx_hbm.at[pl.ds(row0, _RPW)], idx_v)

        for b in range(_NB):
            pltpu.async_copy(
                table_hbm.at[idx_v.at[pl.ds(b * _K, _K)]],
                bufs[b],
                gsem.at[b],
            )

        def body(i, carry):
            for b in range(_NB):
                g = i * _NB + b
                pltpu.make_async_copy(
                    table_hbm.at[idx_v.at[pl.ds(g * _K, _K)]],
                    bufs[b],
                    gsem.at[b],
                ).wait()
                pltpu.async_copy(
                    bufs[b],
                    out_hbm.at[pl.ds(row0 + g * _K, _K)],
                    wsem.at[b],
                )
            for b in range(_NB):
                gnext = jnp.minimum((i + 1) * _NB + b, _CPW - 1)
                pltpu.make_async_copy(
                    bufs[b], out_hbm.at[pl.ds(0, _K)], wsem.at[b]
                ).wait()
                pltpu.async_copy(
                    table_hbm.at[idx_v.at[pl.ds(gnext * _K, _K)]],
                    bufs[b],
                    gsem.at[b],
                )
            return carry

        lax.fori_loop(0, _CPW // _NB, body, 0)

        for b in range(_NB):
            pltpu.make_async_copy(
                table_hbm.at[idx_v.at[pl.ds(0, _K)]],
                bufs[b],
                gsem.at[b],
            ).wait()

    return gather_kernel


_gather = _make_gather()


_TBC = 30720


def _make_padder():
    def body(in_ref, out_ref):
        out_ref[:, :_EMSIZE] = jnp.transpose(in_ref[...])

    return pl.pallas_call(
        body,
        grid=((_VOCAB + _TBC - 1) // _TBC,),
        in_specs=[pl.BlockSpec((_EMSIZE, _TBC), lambda i: (0, i))],
        out_specs=pl.BlockSpec((_TBC, _PADE), lambda i: (i, 0)),
        out_shape=jax.ShapeDtypeStruct((_VOCAB, _PADE), jnp.float32),
    )


_padder = _make_padder()


def kernel(sequence, table):
    idx = sequence.astype(jnp.int32).reshape(_N)
    table_pad = _padder(table.T)
    out = _gather(table_pad, idx)
    return out[:, :_EMSIZE].reshape(_BATCH, _SEQLEN, _EMSIZE)


# speedup vs baseline: 1.5336x; 1.0032x over previous
import functools

import jax
import jax.numpy as jnp
from jax import lax
from jax.experimental import pallas as pl
from jax.experimental.pallas import tpu as pltpu
from jax.experimental.pallas import tpu_sc as plsc

_VOCAB = 1000000
_EMSIZE = 64
_PADE = 128
_BATCH = 4096
_SEQLEN = 200

_N = _BATCH * _SEQLEN

_info = plsc.get_sparse_core_info()
_NC, _NS = _info.num_cores, _info.num_subcores
_NW = _NC * _NS
_RPW = _N // _NW

_K = 256
_NB = 3
_CPW = _RPW // _K


def _make_gather():
    mesh = plsc.VectorSubcoreMesh(core_axis_name="c", subcore_axis_name="s")

    @functools.partial(
        pl.kernel,
        mesh=mesh,
        out_type=jax.ShapeDtypeStruct((_N, _PADE), jnp.float32),
        scratch_types=[
            pltpu.VMEM((_RPW,), jnp.int32),
            [pltpu.VMEM((_K, _PADE), jnp.float32) for _ in range(_NB)],
            pltpu.SemaphoreType.DMA((_NB,)),
            pltpu.SemaphoreType.DMA((_NB,)),
        ],
        compiler_params=pltpu.CompilerParams(use_tc_tiling_on_sc=True),
    )
    def gather_kernel(table_hbm, idx_hbm, out_hbm, idx_v, bufs, gsem, wsem):
        wid = lax.axis_index("s") * _NC + lax.axis_index("c")
        row0 = wid * _RPW
        pltpu.sync_copy(idx_hbm.at[pl.ds(row0, _RPW)], idx_v)

        for b in range(_NB):
            pltpu.async_copy(
                table_hbm.at[idx_v.at[pl.ds(b * _K, _K)]],
                bufs[b],
                gsem.at[b],
            )

        def body(i, carry):
            for b in range(_NB):
                g = i * _NB + b
                pltpu.make_async_copy(
                    table_hbm.at[idx_v.at[pl.ds(g * _K, _K)]],
                    bufs[b],
                    gsem.at[b],
                ).wait()
                pltpu.async_copy(
                    bufs[b],
                    out_hbm.at[pl.ds(row0 + g * _K, _K)],
                    wsem.at[b],
                )
            for b in range(_NB):
                gnext = jnp.minimum((i + 1) * _NB + b, _CPW - 1)
                pltpu.make_async_copy(
                    bufs[b], out_hbm.at[pl.ds(0, _K)], wsem.at[b]
                ).wait()
                pltpu.async_copy(
                    table_hbm.at[idx_v.at[pl.ds(gnext * _K, _K)]],
                    bufs[b],
                    gsem.at[b],
                )
            return carry

        lax.fori_loop(0, _CPW // _NB, body, 0)

        for b in range(_NB):
            pltpu.make_async_copy(
                table_hbm.at[idx_v.at[pl.ds(0, _K)]],
                bufs[b],
                gsem.at[b],
            ).wait()

    return gather_kernel


_gather = _make_gather()


_TBC = 30720


def _make_padder():
    def body(in_ref, out_ref):
        out_ref[:, :_EMSIZE] = jnp.transpose(in_ref[...])

    return pl.pallas_call(
        body,
        grid=((_VOCAB + _TBC - 1) // _TBC,),
        in_specs=[pl.BlockSpec((_EMSIZE, _TBC), lambda i: (0, i))],
        out_specs=pl.BlockSpec((_TBC, _PADE), lambda i: (i, 0)),
        out_shape=jax.ShapeDtypeStruct((_VOCAB, _PADE), jnp.float32),
    )


_padder = _make_padder()


def kernel(sequence, table):
    idx = sequence.astype(jnp.int32).reshape(_N)
    table_pad = _padder(table.T)
    out = _gather(table_pad, idx)
    return out[:, :_EMSIZE].reshape(_BATCH, _SEQLEN, _EMSIZE)


# speedup vs baseline: 1.5374x; 1.0025x over previous
import functools

import jax
import jax.numpy as jnp
from jax import lax
from jax.experimental import pallas as pl
from jax.experimental.pallas import tpu as pltpu
from jax.experimental.pallas import tpu_sc as plsc

_VOCAB = 1000000
_EMSIZE = 64
_PADE = 128
_BATCH = 4096
_SEQLEN = 200

_N = _BATCH * _SEQLEN

_info = plsc.get_sparse_core_info()
_NC, _NS = _info.num_cores, _info.num_subcores
_NW = _NC * _NS
_RPW = _N // _NW

_K = 256
_NB = 3
_CPW = _RPW // _K


def _make_gather():
    mesh = plsc.VectorSubcoreMesh(core_axis_name="c", subcore_axis_name="s")

    @functools.partial(
        pl.kernel,
        mesh=mesh,
        out_type=jax.ShapeDtypeStruct((_N, _PADE), jnp.float32),
        scratch_types=[
            pltpu.VMEM((_RPW,), jnp.int32),
            [pltpu.VMEM((_K, _PADE), jnp.float32) for _ in range(_NB)],
            pltpu.SemaphoreType.DMA((_NB,)),
            pltpu.SemaphoreType.DMA((_NB,)),
        ],
        compiler_params=pltpu.CompilerParams(use_tc_tiling_on_sc=True),
    )
    def gather_kernel(table_hbm, idx_hbm, out_hbm, idx_v, bufs, gsem, wsem):
        wid = lax.axis_index("s") * _NC + lax.axis_index("c")
        row0 = wid * _RPW
        pltpu.sync_copy(idx_hbm.at[pl.ds(row0, _RPW)], idx_v)

        for b in range(_NB):
            pltpu.async_copy(
                table_hbm.at[idx_v.at[pl.ds(b * _K, _K)]],
                bufs[b],
                gsem.at[b],
            )

        def body(i, carry):
            for b in range(_NB):
                g = i * _NB + b
                pltpu.make_async_copy(
                    table_hbm.at[idx_v.at[pl.ds(g * _K, _K)]],
                    bufs[b],
                    gsem.at[b],
                ).wait()
                pltpu.async_copy(
                    bufs[b],
                    out_hbm.at[pl.ds(row0 + g * _K, _K)],
                    wsem.at[b],
                )
            for b in range(_NB):
                gnext = jnp.minimum((i + 1) * _NB + b, _CPW - 1)
                pltpu.make_async_copy(
                    bufs[b], out_hbm.at[pl.ds(0, _K)], wsem.at[b]
                ).wait()
                pltpu.async_copy(
                    table_hbm.at[idx_v.at[pl.ds(gnext * _K, _K)]],
                    bufs[b],
                    gsem.at[b],
                )
            return carry

        lax.fori_loop(0, _CPW // _NB, body, 0)

        for b in range(_NB):
            pltpu.make_async_copy(
                table_hbm.at[idx_v.at[pl.ds(0, _K)]],
                bufs[b],
                gsem.at[b],
            ).wait()

    return gather_kernel


_gather = _make_gather()


_TBC = 38400


def _make_padder():
    def body(in_ref, out_ref):
        out_ref[:, :_EMSIZE] = jnp.transpose(in_ref[...])

    return pl.pallas_call(
        body,
        grid=((_VOCAB + _TBC - 1) // _TBC,),
        in_specs=[pl.BlockSpec((_EMSIZE, _TBC), lambda i: (0, i))],
        out_specs=pl.BlockSpec((_TBC, _PADE), lambda i: (i, 0)),
        out_shape=jax.ShapeDtypeStruct((_VOCAB, _PADE), jnp.float32),
        compiler_params=pltpu.CompilerParams(vmem_limit_bytes=128 * 1024 * 1024),
    )


_padder = _make_padder()


def kernel(sequence, table):
    idx = sequence.astype(jnp.int32).reshape(_N)
    table_pad = _padder(table.T)
    out = _gather(table_pad, idx)
    return out[:, :_EMSIZE].reshape(_BATCH, _SEQLEN, _EMSIZE)
